# baseline (speedup 1.0000x reference)
import functools

import jax
import jax.numpy as jnp
from jax import lax
from jax.experimental import pallas as pl
from jax.experimental.pallas import tpu as pltpu
from jax.experimental.pallas import tpu_sc as plsc


def _dist_kernel(x_ref, dis_ref, at0_ref, solo0_ref):
    x = x_ref[...]
    BB, S, C = x.shape
    INF = jnp.float32(jnp.inf)
    BIG = jnp.int32(10 ** 9)

    sq = jnp.sum(x * x, axis=-1)
    G = jnp.stack([
        lax.dot_general(x[s], x[s], (((1,), (1,)), ((), ())),
                        preferred_element_type=jnp.float32)
        for s in range(BB)
    ], axis=0)
    D2 = (sq[:, :, None] + sq[:, None, :]) - 2.0 * G
    Dis = jnp.sqrt(jnp.maximum(D2, 0.0))

    ii = lax.broadcasted_iota(jnp.int32, (BB, S), 1)
    eye = (lax.broadcasted_iota(jnp.int32, (S, S), 0) ==
           lax.broadcasted_iota(jnp.int32, (S, S), 1))

    m = jnp.where(eye[None], INF, Dis)
    rowmin = jnp.min(m, axis=2)
    gmin = jnp.min(rowmin, axis=1, keepdims=True)
    i0 = jnp.min(jnp.where(rowmin == gmin, ii, BIG), axis=1, keepdims=True)
    ohf_i0 = jnp.where(ii == i0, 1.0, 0.0)
    row_i0 = jnp.sum(jnp.where(ohf_i0[:, None, :] == 1.0, m, 0.0), axis=2)
    j0 = jnp.min(jnp.where(row_i0 == gmin, ii, BIG), axis=1, keepdims=True)
    ohf_j0 = jnp.where(ii == j0, 1.0, 0.0)

    dis_ref[...] = Dis
    at0_ref[...] = (ohf_i0[:, :, None] * ohf_j0[:, None, :] +
                    ohf_j0[:, :, None] * ohf_i0[:, None, :])
    solo0_ref[...] = 1.0 - jnp.maximum(ohf_i0, ohf_j0)


def _make_sc_builder(B, S):
    info = plsc.get_sparse_core_info()
    NC, NS = info.num_cores, info.num_subcores
    NW = NC * NS
    assert B % NW == 0
    SPW = B // NW
    NCH = S // 16
    mesh = plsc.VectorSubcoreMesh(core_axis_name="c", subcore_axis_name="s")

    @functools.partial(
        pl.kernel,
        out_type=jax.ShapeDtypeStruct((B, S, S), jnp.float32),
        mesh=mesh,
        scratch_types=[
            pltpu.VMEM((S, S), jnp.float32),
            pltpu.VMEM((S, S), jnp.float32),
            pltpu.VMEM((S,), jnp.float32),
        ],
        compiler_params=pltpu.CompilerParams(needs_layout_passes=False),
    )
    def builder(dis_hbm, at0_hbm, solo0_hbm, at_hbm, dis_v, at_v, solo_v):
        wid = lax.axis_index("s") * NC + lax.axis_index("c")
        iota = lax.iota(jnp.int32, 16)
        cols = [iota + 16 * c for c in range(NCH)]
        perms = [iota ^ k for k in (8, 4, 2, 1)]
        INF = jnp.float32(jnp.inf)
        BIGI = jnp.int32(10 ** 9)

        def lanes_min(v):
            for p in perms:
                v = jnp.minimum(v, v.at[p].get(mode="promise_in_bounds"))
            return v

        def lanes_sum(v):
            for p in perms:
                v = v + v.at[p].get(mode="promise_in_bounds")
            return v

        lane_splats = [jnp.full((16,), l, dtype=jnp.int32) for l in range(16)]

        def per_sample(s, carry):
            b = wid * SPW + s
            pltpu.sync_copy(dis_hbm.at[b], dis_v)
            pltpu.sync_copy(at0_hbm.at[b], at_v)
            pltpu.sync_copy(solo0_hbm.at[b], solo_v)
            solo_init = tuple(solo_v[pl.ds(16 * c, 16)] for c in range(NCH))

            def round_body(t, carry2):
                solo_ch = list(carry2)
                clu_ch = [1.0 - v for v in solo_ch]
                solo_b = [v == 1.0 for v in solo_ch]
                ncv = lanes_sum(clu_ch[0] + clu_ch[1] + clu_ch[2] + clu_ch[3])
                nsv = jnp.maximum(jnp.float32(S) - ncv, 1.0)

                part = [[jnp.zeros((16,), jnp.float32)] * (2 * NCH)
                        for _ in range(NCH)]
                for cj_blk in range(NCH):
                    soloc = solo_ch[cj_blk]
                    pacc = part[cj_blk]
                    for l in range(16):
                        j = 16 * cj_blk + l
                        sj = soloc.at[lane_splats[l]].get(
                            mode="promise_in_bounds")
                        cj = 1.0 - sj
                        for c in range(NCH):
                            row = dis_v[j, pl.ds(16 * c, 16)]
                            pacc[c] = pacc[c] + row * cj
                            pacc[NCH + c] = pacc[NCH + c] + row * sj
                acc = [(part[0][k] + part[1][k]) + (part[2][k] + part[3][k])
                       for k in range(2 * NCH)]
                ms2c = [jnp.where(solo_b[c], acc[c] / ncv, INF)
                        for c in range(NCH)]
                mc2s = [jnp.where(solo_b[c], INF, acc[NCH + c] / nsv)
                        for c in range(NCH)]

                def two_smallest(vch):
                    v1 = lanes_min(jnp.minimum(jnp.minimum(vch[0], vch[1]),
                                               jnp.minimum(vch[2], vch[3])))
                    cand = [jnp.where(vch[c] == v1, cols[c], BIGI)
                            for c in range(NCH)]
                    am = lanes_min(jnp.minimum(jnp.minimum(cand[0], cand[1]),
                                               jnp.minimum(cand[2], cand[3])))
                    masked = [jnp.where(cols[c] == am, INF, vch[c])
                              for c in range(NCH)]
                    v2 = lanes_min(jnp.minimum(jnp.minimum(masked[0], masked[1]),
                                               jnp.minimum(masked[2], masked[3])))
                    return v1, am, v2

                v1, ssp, v2 = two_smallest(ms2c)
                _, _, w2 = two_smallest(mc2s)
                tv = jnp.full((16,), t, dtype=jnp.int32)
                t1v = jnp.where(tv < S - 3, v2, v1)
                t2v = w2
                g1v = t1v < INF
                g2v = t2v < INF
                e_rows = []
                e_cols = []
                ones = jnp.ones((16,), jnp.float32)
                for c in range(NCH):
                    d_c = plsc.load_gather(dis_v, [ssp, cols[c]])
                    e_row = ((d_c - t1v) < 0.0) & g1v & ~solo_b[c]
                    e_col = ((d_c - t2v) < 0.0) & g2v & ~solo_b[c]
                    plsc.store_scatter(at_v, [ssp, cols[c]], ones, mask=e_row)
                    plsc.store_scatter(at_v, [cols[c], ssp], ones, mask=e_col)
                    e_rows.append(e_row)
                    e_cols.append(e_col)
                anyf = jnp.float32(0.0)
                for c in range(NCH):
                    anyf = anyf + jnp.where(e_rows[c] | e_cols[c], 1.0, 0.0)
                keepf = jnp.where(lanes_sum(anyf) > 0.0, 0.0, 1.0)
                return tuple(
                    jnp.where(cols[c] == ssp, solo_ch[c] * keepf, solo_ch[c])
                    for c in range(NCH))

            lax.fori_loop(0, S - 2, round_body, solo_init)
            pltpu.sync_copy(at_v, at_hbm.at[b])
            return carry

        lax.fori_loop(0, SPW, per_sample, 0)

    return builder


def _out_kernel(at_ref, x_ref, W1_ref, W2_ref, out_ref):
    BB = at_ref.shape[0]
    W1 = W1_ref[...]
    W2 = W2_ref[...]
    for s in range(BB):
        acc = lax.dot_general(at_ref[s], x_ref[s], (((1,), (0,)), ((), ())),
                              preferred_element_type=jnp.float32)
        h = lax.dot_general(acc, W1, (((1,), (0,)), ((), ())),
                            preferred_element_type=jnp.float32)
        out_ref[s] = jnp.maximum(h + W2, 0.0)


def _halves(x, W1, W2, n_chunks):
    B, S, C = x.shape
    BB = 8
    BH = B // n_chunks
    builder = _make_sc_builder(BH, S)
    outs = []
    for h in range(n_chunks):
        xh = lax.slice_in_dim(x, h * BH, (h + 1) * BH, axis=0)
        dis, at0, solo0 = pl.pallas_call(
            _dist_kernel,
            grid=(BH // BB,),
            in_specs=[pl.BlockSpec((BB, S, C), lambda i: (i, 0, 0))],
            out_specs=[
                pl.BlockSpec((BB, S, S), lambda i: (i, 0, 0)),
                pl.BlockSpec((BB, S, S), lambda i: (i, 0, 0)),
                pl.BlockSpec((BB, S), lambda i: (i, 0)),
            ],
            out_shape=[
                jax.ShapeDtypeStruct((BH, S, S), jnp.float32),
                jax.ShapeDtypeStruct((BH, S, S), jnp.float32),
                jax.ShapeDtypeStruct((BH, S), jnp.float32),
            ],
        )(xh)
        at = builder(dis, at0, solo0)
        outs.append(pl.pallas_call(
            _out_kernel,
            grid=(BH // BB,),
            in_specs=[
                pl.BlockSpec((BB, S, S), lambda i: (i, 0, 0)),
                pl.BlockSpec((BB, S, C), lambda i: (i, 0, 0)),
                pl.BlockSpec((C, C), lambda i: (0, 0)),
                pl.BlockSpec((S, C), lambda i: (0, 0)),
            ],
            out_specs=pl.BlockSpec((BB, S, C), lambda i: (i, 0, 0)),
            out_shape=jax.ShapeDtypeStruct((BH, S, C), x.dtype),
        )(at, xh, W1, W2))
    return jnp.concatenate(outs, axis=0)


@jax.jit
def kernel(x, W1, W2):
    return _halves(x, W1, W2, 2)


# speedup vs baseline: 1.0075x; 1.0075x over previous
import functools

import jax
import jax.numpy as jnp
from jax import lax
from jax.experimental import pallas as pl
from jax.experimental.pallas import tpu as pltpu
from jax.experimental.pallas import tpu_sc as plsc


def _dist_kernel(x_ref, dis_ref, at0_ref, solo0_ref):
    x = x_ref[...]
    BB, S, C = x.shape
    INF = jnp.float32(jnp.inf)
    BIG = jnp.int32(10 ** 9)

    sq = jnp.sum(x * x, axis=-1)
    G = jnp.stack([
        lax.dot_general(x[s], x[s], (((1,), (1,)), ((), ())),
                        preferred_element_type=jnp.float32)
        for s in range(BB)
    ], axis=0)
    D2 = (sq[:, :, None] + sq[:, None, :]) - 2.0 * G
    Dis = jnp.sqrt(jnp.maximum(D2, 0.0))

    ii = lax.broadcasted_iota(jnp.int32, (BB, S), 1)
    eye = (lax.broadcasted_iota(jnp.int32, (S, S), 0) ==
           lax.broadcasted_iota(jnp.int32, (S, S), 1))

    m = jnp.where(eye[None], INF, Dis)
    rowmin = jnp.min(m, axis=2)
    gmin = jnp.min(rowmin, axis=1, keepdims=True)
    i0 = jnp.min(jnp.where(rowmin == gmin, ii, BIG), axis=1, keepdims=True)
    ohf_i0 = jnp.where(ii == i0, 1.0, 0.0)
    row_i0 = jnp.sum(jnp.where(ohf_i0[:, None, :] == 1.0, m, 0.0), axis=2)
    j0 = jnp.min(jnp.where(row_i0 == gmin, ii, BIG), axis=1, keepdims=True)
    ohf_j0 = jnp.where(ii == j0, 1.0, 0.0)

    dis_ref[...] = Dis
    at0_ref[...] = (ohf_i0[:, :, None] * ohf_j0[:, None, :] +
                    ohf_j0[:, :, None] * ohf_i0[:, None, :])
    solo0_ref[...] = 1.0 - jnp.maximum(ohf_i0, ohf_j0)


def _make_sc_builder(B, S):
    info = plsc.get_sparse_core_info()
    NC, NS = info.num_cores, info.num_subcores
    NW = NC * NS
    assert B % NW == 0
    SPW = B // NW
    NCH = S // 16
    mesh = plsc.VectorSubcoreMesh(core_axis_name="c", subcore_axis_name="s")

    assert SPW == 2, "builder interleaves exactly two samples per subcore"

    @functools.partial(
        pl.kernel,
        out_type=jax.ShapeDtypeStruct((B, S, S), jnp.float32),
        mesh=mesh,
        scratch_types=[
            pltpu.VMEM((2, S, S), jnp.float32),
            pltpu.VMEM((2, S, S), jnp.float32),
            pltpu.VMEM((2, S), jnp.float32),
        ],
        compiler_params=pltpu.CompilerParams(needs_layout_passes=False),
    )
    def builder(dis_hbm, at0_hbm, solo0_hbm, at_hbm, dis_v, at_v, solo_v):
        wid = lax.axis_index("s") * NC + lax.axis_index("c")
        iota = lax.iota(jnp.int32, 16)
        cols = [iota + 16 * c for c in range(NCH)]
        perms = [iota ^ k for k in (8, 4, 2, 1)]
        INF = jnp.float32(jnp.inf)
        BIGI = jnp.int32(10 ** 9)

        def lanes_min(v):
            for p in perms:
                v = jnp.minimum(v, v.at[p].get(mode="promise_in_bounds"))
            return v

        def lanes_sum(v):
            for p in perms:
                v = v + v.at[p].get(mode="promise_in_bounds")
            return v

        lane_splats = [jnp.full((16,), l, dtype=jnp.int32) for l in range(16)]

        def one_round(p, t, solo_ch, ncv):
            clu_ch = [1.0 - v for v in solo_ch]
            solo_b = [v == 1.0 for v in solo_ch]
            nsv = jnp.maximum(jnp.float32(S) - ncv, 1.0)

            acc = [jnp.zeros((16,), jnp.float32)] * (2 * NCH)
            for cj_blk in range(NCH):
                soloc = solo_ch[cj_blk]
                for l in range(16):
                    j = 16 * cj_blk + l
                    sj = soloc.at[lane_splats[l]].get(mode="promise_in_bounds")
                    cj = 1.0 - sj
                    for c in range(NCH):
                        row = dis_v[p, j, pl.ds(16 * c, 16)]
                        acc[c] = acc[c] + row * cj
                        acc[NCH + c] = acc[NCH + c] + row * sj
            ms2c = [jnp.where(solo_b[c], acc[c] / ncv, INF)
                    for c in range(NCH)]
            mc2s = [jnp.where(solo_b[c], INF, acc[NCH + c] / nsv)
                    for c in range(NCH)]

            def two_smallest(vch):
                v1 = lanes_min(jnp.minimum(jnp.minimum(vch[0], vch[1]),
                                           jnp.minimum(vch[2], vch[3])))
                cand = [jnp.where(vch[c] == v1, cols[c], BIGI)
                        for c in range(NCH)]
                am = lanes_min(jnp.minimum(jnp.minimum(cand[0], cand[1]),
                                           jnp.minimum(cand[2], cand[3])))
                masked = [jnp.where(cols[c] == am, INF, vch[c])
                          for c in range(NCH)]
                v2 = lanes_min(jnp.minimum(jnp.minimum(masked[0], masked[1]),
                                           jnp.minimum(masked[2], masked[3])))
                return v1, am, v2

            v1, ssp, v2 = two_smallest(ms2c)
            _, _, w2 = two_smallest(mc2s)
            tv = jnp.full((16,), t, dtype=jnp.int32)
            t1v = jnp.where(tv < S - 3, v2, v1)
            t2v = w2
            g1v = t1v < INF
            g2v = t2v < INF
            psp = lane_splats[p]
            ones = jnp.ones((16,), jnp.float32)
            anyf = jnp.float32(0.0)
            for c in range(NCH):
                d_c = plsc.load_gather(dis_v, [psp, ssp, cols[c]])
                e_row = ((d_c - t1v) < 0.0) & g1v & ~solo_b[c]
                e_col = ((d_c - t2v) < 0.0) & g2v & ~solo_b[c]
                plsc.store_scatter(at_v, [psp, ssp, cols[c]], ones, mask=e_row)
                plsc.store_scatter(at_v, [psp, cols[c], ssp], ones, mask=e_col)
                anyf = anyf + jnp.where(e_row | e_col, 1.0, 0.0)
            addf = jnp.where(lanes_sum(anyf) > 0.0, 1.0, 0.0)
            new_solo = tuple(
                jnp.where(cols[c] == ssp, solo_ch[c] * (1.0 - addf),
                          solo_ch[c])
                for c in range(NCH))
            return new_solo, ncv + addf

        b0 = wid * SPW
        pltpu.sync_copy(dis_hbm.at[b0], dis_v.at[0])
        pltpu.sync_copy(dis_hbm.at[b0 + 1], dis_v.at[1])
        pltpu.sync_copy(at0_hbm.at[b0], at_v.at[0])
        pltpu.sync_copy(at0_hbm.at[b0 + 1], at_v.at[1])
        pltpu.sync_copy(solo0_hbm.at[b0], solo_v.at[0])
        pltpu.sync_copy(solo0_hbm.at[b0 + 1], solo_v.at[1])
        init = (tuple(solo_v[0, pl.ds(16 * c, 16)] for c in range(NCH)) +
                tuple(solo_v[1, pl.ds(16 * c, 16)] for c in range(NCH)) +
                (jnp.full((16,), 2.0), jnp.full((16,), 2.0)))

        def round_body(t, carry):
            s0 = list(carry[:NCH])
            s1 = list(carry[NCH:2 * NCH])
            nc0 = carry[2 * NCH]
            nc1 = carry[2 * NCH + 1]
            new0, nc0 = one_round(0, t, s0, nc0)
            new1, nc1 = one_round(1, t, s1, nc1)
            return new0 + new1 + (nc0, nc1)

        lax.fori_loop(0, S - 2, round_body, init)
        pltpu.sync_copy(at_v.at[0], at_hbm.at[b0])
        pltpu.sync_copy(at_v.at[1], at_hbm.at[b0 + 1])

    return builder


def _out_kernel(at_ref, x_ref, W1_ref, W2_ref, out_ref):
    BB = at_ref.shape[0]
    W1 = W1_ref[...]
    W2 = W2_ref[...]
    for s in range(BB):
        acc = lax.dot_general(at_ref[s], x_ref[s], (((1,), (0,)), ((), ())),
                              preferred_element_type=jnp.float32)
        h = lax.dot_general(acc, W1, (((1,), (0,)), ((), ())),
                            preferred_element_type=jnp.float32)
        out_ref[s] = jnp.maximum(h + W2, 0.0)


def _halves(x, W1, W2, n_chunks):
    B, S, C = x.shape
    BB = 8
    BH = B // n_chunks
    builder = _make_sc_builder(BH, S)
    outs = []
    for h in range(n_chunks):
        xh = lax.slice_in_dim(x, h * BH, (h + 1) * BH, axis=0)
        dis, at0, solo0 = pl.pallas_call(
            _dist_kernel,
            grid=(BH // BB,),
            in_specs=[pl.BlockSpec((BB, S, C), lambda i: (i, 0, 0))],
            out_specs=[
                pl.BlockSpec((BB, S, S), lambda i: (i, 0, 0)),
                pl.BlockSpec((BB, S, S), lambda i: (i, 0, 0)),
                pl.BlockSpec((BB, S), lambda i: (i, 0)),
            ],
            out_shape=[
                jax.ShapeDtypeStruct((BH, S, S), jnp.float32),
                jax.ShapeDtypeStruct((BH, S, S), jnp.float32),
                jax.ShapeDtypeStruct((BH, S), jnp.float32),
            ],
        )(xh)
        at = builder(dis, at0, solo0)
        outs.append(pl.pallas_call(
            _out_kernel,
            grid=(BH // BB,),
            in_specs=[
                pl.BlockSpec((BB, S, S), lambda i: (i, 0, 0)),
                pl.BlockSpec((BB, S, C), lambda i: (i, 0, 0)),
                pl.BlockSpec((C, C), lambda i: (0, 0)),
                pl.BlockSpec((S, C), lambda i: (0, 0)),
            ],
            out_specs=pl.BlockSpec((BB, S, C), lambda i: (i, 0, 0)),
            out_shape=jax.ShapeDtypeStruct((BH, S, C), x.dtype),
        )(at, xh, W1, W2))
    return jnp.concatenate(outs, axis=0)


@jax.jit
def kernel(x, W1, W2):
    return _halves(x, W1, W2, 2)


# speedup vs baseline: 1.0588x; 1.0509x over previous
import functools

import jax
import jax.numpy as jnp
from jax import lax
from jax.experimental import pallas as pl
from jax.experimental.pallas import tpu as pltpu
from jax.experimental.pallas import tpu_sc as plsc


def _dist_kernel(x_ref, dis_ref, at0_ref, solo0_ref):
    x = x_ref[...]
    BB, S, C = x.shape
    INF = jnp.float32(jnp.inf)
    BIG = jnp.int32(10 ** 9)

    sq = jnp.sum(x * x, axis=-1)
    G = jnp.stack([
        lax.dot_general(x[s], x[s], (((1,), (1,)), ((), ())),
                        preferred_element_type=jnp.float32)
        for s in range(BB)
    ], axis=0)
    D2 = (sq[:, :, None] + sq[:, None, :]) - 2.0 * G
    Dis = jnp.sqrt(jnp.maximum(D2, 0.0))

    ii = lax.broadcasted_iota(jnp.int32, (BB, S), 1)
    eye = (lax.broadcasted_iota(jnp.int32, (S, S), 0) ==
           lax.broadcasted_iota(jnp.int32, (S, S), 1))

    m = jnp.where(eye[None], INF, Dis)
    rowmin = jnp.min(m, axis=2)
    gmin = jnp.min(rowmin, axis=1, keepdims=True)
    i0 = jnp.min(jnp.where(rowmin == gmin, ii, BIG), axis=1, keepdims=True)
    ohf_i0 = jnp.where(ii == i0, 1.0, 0.0)
    row_i0 = jnp.sum(jnp.where(ohf_i0[:, None, :] == 1.0, m, 0.0), axis=2)
    j0 = jnp.min(jnp.where(row_i0 == gmin, ii, BIG), axis=1, keepdims=True)
    ohf_j0 = jnp.where(ii == j0, 1.0, 0.0)

    dis_ref[...] = Dis
    at0_ref[...] = (ohf_i0[:, :, None] * ohf_j0[:, None, :] +
                    ohf_j0[:, :, None] * ohf_i0[:, None, :])
    solo0_ref[...] = 1.0 - jnp.maximum(ohf_i0, ohf_j0)


def _make_sc_builder(B, S):
    info = plsc.get_sparse_core_info()
    NC, NS = info.num_cores, info.num_subcores
    NW = NC * NS
    assert B % NW == 0
    SPW = B // NW
    NCH = S // 16
    mesh = plsc.VectorSubcoreMesh(core_axis_name="c", subcore_axis_name="s")

    @functools.partial(
        pl.kernel,
        out_type=jax.ShapeDtypeStruct((B, S, S), jnp.float32),
        mesh=mesh,
        scratch_types=[
            pltpu.VMEM((S, S), jnp.float32),
            pltpu.VMEM((S, S), jnp.float32),
            pltpu.VMEM((S,), jnp.float32),
        ],
        compiler_params=pltpu.CompilerParams(needs_layout_passes=False),
    )
    def builder(dis_hbm, at0_hbm, solo0_hbm, at_hbm, dis_v, at_v, solo_v):
        wid = lax.axis_index("s") * NC + lax.axis_index("c")
        iota = lax.iota(jnp.int32, 16)
        cols = [iota + 16 * c for c in range(NCH)]
        perms = [iota ^ k for k in (8, 4, 2, 1)]
        INF = jnp.float32(jnp.inf)
        BIGI = jnp.int32(10 ** 9)

        def lanes_min(v):
            for p in perms:
                v = jnp.minimum(v, v.at[p].get(mode="promise_in_bounds"))
            return v

        def lanes_sum(v):
            for p in perms:
                v = v + v.at[p].get(mode="promise_in_bounds")
            return v

        lane_splats = [jnp.full((16,), l, dtype=jnp.int32) for l in range(16)]

        def per_sample(s, carry):
            b = wid * SPW + s
            pltpu.sync_copy(dis_hbm.at[b], dis_v)
            pltpu.sync_copy(at0_hbm.at[b], at_v)
            pltpu.sync_copy(solo0_hbm.at[b], solo_v)
            init = (tuple(solo_v[pl.ds(16 * c, 16)] for c in range(NCH)) +
                    (jnp.full((16,), 2.0),))

            def round_body(t, carry2):
                solo_ch = list(carry2[:NCH])
                ncv = carry2[NCH]
                solo_b = [v == 1.0 for v in solo_ch]
                nsv = jnp.maximum(jnp.float32(S) - ncv, 1.0)

                acc = [jnp.zeros((16,), jnp.float32)] * (2 * NCH)
                for cj_blk in range(NCH):
                    soloc = solo_ch[cj_blk]
                    for l in range(16):
                        j = 16 * cj_blk + l
                        sj = soloc.at[lane_splats[l]].get(
                            mode="promise_in_bounds")
                        cj = 1.0 - sj
                        for c in range(NCH):
                            row = dis_v[j, pl.ds(16 * c, 16)]
                            acc[c] = acc[c] + row * cj
                            acc[NCH + c] = acc[NCH + c] + row * sj
                ms2c = [jnp.where(solo_b[c], acc[c] / ncv, INF)
                        for c in range(NCH)]
                mc2s = [jnp.where(solo_b[c], INF, acc[NCH + c] / nsv)
                        for c in range(NCH)]

                def two_smallest(vch):
                    v1 = lanes_min(jnp.minimum(jnp.minimum(vch[0], vch[1]),
                                               jnp.minimum(vch[2], vch[3])))
                    cand = [jnp.where(vch[c] == v1, cols[c], BIGI)
                            for c in range(NCH)]
                    am = lanes_min(jnp.minimum(jnp.minimum(cand[0], cand[1]),
                                               jnp.minimum(cand[2], cand[3])))
                    masked = [jnp.where(cols[c] == am, INF, vch[c])
                              for c in range(NCH)]
                    v2 = lanes_min(jnp.minimum(
                        jnp.minimum(masked[0], masked[1]),
                        jnp.minimum(masked[2], masked[3])))
                    return v1, am, v2

                v1, ssp, v2 = two_smallest(ms2c)
                _, _, w2 = two_smallest(mc2s)
                tv = jnp.full((16,), t, dtype=jnp.int32)
                t1v = jnp.where(tv < S - 3, v2, v1)
                t2v = w2
                g1v = t1v < INF
                g2v = t2v < INF
                ones = jnp.ones((16,), jnp.float32)
                anyf = jnp.float32(0.0)
                for c in range(NCH):
                    d_c = plsc.load_gather(dis_v, [ssp, cols[c]])
                    e_row = ((d_c - t1v) < 0.0) & g1v & ~solo_b[c]
                    e_col = ((d_c - t2v) < 0.0) & g2v & ~solo_b[c]
                    plsc.store_scatter(at_v, [ssp, cols[c]], ones,
                                       mask=e_row)
                    plsc.store_scatter(at_v, [cols[c], ssp], ones,
                                       mask=e_col)
                    anyf = anyf + jnp.where(e_row | e_col, 1.0, 0.0)
                addf = jnp.where(lanes_sum(anyf) > 0.0, 1.0, 0.0)
                new_solo = tuple(
                    jnp.where(cols[c] == ssp, solo_ch[c] * (1.0 - addf),
                              solo_ch[c])
                    for c in range(NCH))
                return new_solo + (ncv + addf,)

            lax.fori_loop(0, S - 2, round_body, init)
            pltpu.sync_copy(at_v, at_hbm.at[b])
            return carry

        lax.fori_loop(0, SPW, per_sample, 0)

    return builder


def _out_kernel(at_ref, x_ref, W1_ref, W2_ref, out_ref):
    BB = at_ref.shape[0]
    W1 = W1_ref[...]
    W2 = W2_ref[...]
    for s in range(BB):
        acc = lax.dot_general(at_ref[s], x_ref[s], (((1,), (0,)), ((), ())),
                              preferred_element_type=jnp.float32)
        h = lax.dot_general(acc, W1, (((1,), (0,)), ((), ())),
                            preferred_element_type=jnp.float32)
        out_ref[s] = jnp.maximum(h + W2, 0.0)


def _halves(x, W1, W2, n_chunks):
    B, S, C = x.shape
    BB = 16
    BH = B // n_chunks
    builder = _make_sc_builder(BH, S)
    outs = []
    for h in range(n_chunks):
        xh = lax.slice_in_dim(x, h * BH, (h + 1) * BH, axis=0)
        dis, at0, solo0 = pl.pallas_call(
            _dist_kernel,
            grid=(BH // BB,),
            in_specs=[pl.BlockSpec((BB, S, C), lambda i: (i, 0, 0))],
            out_specs=[
                pl.BlockSpec((BB, S, S), lambda i: (i, 0, 0)),
                pl.BlockSpec((BB, S, S), lambda i: (i, 0, 0)),
                pl.BlockSpec((BB, S), lambda i: (i, 0)),
            ],
            out_shape=[
                jax.ShapeDtypeStruct((BH, S, S), jnp.float32),
                jax.ShapeDtypeStruct((BH, S, S), jnp.float32),
                jax.ShapeDtypeStruct((BH, S), jnp.float32),
            ],
        )(xh)
        at = builder(dis, at0, solo0)
        outs.append(pl.pallas_call(
            _out_kernel,
            grid=(BH // BB,),
            in_specs=[
                pl.BlockSpec((BB, S, S), lambda i: (i, 0, 0)),
                pl.BlockSpec((BB, S, C), lambda i: (i, 0, 0)),
                pl.BlockSpec((C, C), lambda i: (0, 0)),
                pl.BlockSpec((S, C), lambda i: (0, 0)),
            ],
            out_specs=pl.BlockSpec((BB, S, C), lambda i: (i, 0, 0)),
            out_shape=jax.ShapeDtypeStruct((BH, S, C), x.dtype),
        )(at, xh, W1, W2))
    return jnp.concatenate(outs, axis=0)


@jax.jit
def kernel(x, W1, W2):
    return _halves(x, W1, W2, 2)


# speedup vs baseline: 1.1165x; 1.0545x over previous
import functools

import jax
import jax.numpy as jnp
from jax import lax
from jax.experimental import pallas as pl
from jax.experimental.pallas import tpu as pltpu
from jax.experimental.pallas import tpu_sc as plsc


def _dist_kernel(x_ref, dis_ref, at0_ref, solo0_ref):
    x = x_ref[...]
    BB, S, C = x.shape
    INF = jnp.float32(jnp.inf)
    BIG = jnp.int32(10 ** 9)

    sq = jnp.sum(x * x, axis=-1)
    G = jnp.stack([
        lax.dot_general(x[s], x[s], (((1,), (1,)), ((), ())),
                        preferred_element_type=jnp.float32)
        for s in range(BB)
    ], axis=0)
    D2 = (sq[:, :, None] + sq[:, None, :]) - 2.0 * G
    Dis = jnp.sqrt(jnp.maximum(D2, 0.0))

    ii = lax.broadcasted_iota(jnp.int32, (BB, S), 1)
    eye = (lax.broadcasted_iota(jnp.int32, (S, S), 0) ==
           lax.broadcasted_iota(jnp.int32, (S, S), 1))

    m = jnp.where(eye[None], INF, Dis)
    rowmin = jnp.min(m, axis=2)
    gmin = jnp.min(rowmin, axis=1, keepdims=True)
    i0 = jnp.min(jnp.where(rowmin == gmin, ii, BIG), axis=1, keepdims=True)
    ohf_i0 = jnp.where(ii == i0, 1.0, 0.0)
    row_i0 = jnp.sum(jnp.where(ohf_i0[:, None, :] == 1.0, m, 0.0), axis=2)
    j0 = jnp.min(jnp.where(row_i0 == gmin, ii, BIG), axis=1, keepdims=True)
    ohf_j0 = jnp.where(ii == j0, 1.0, 0.0)

    dis_ref[...] = Dis
    at0_ref[...] = (ohf_i0[:, :, None] * ohf_j0[:, None, :] +
                    ohf_j0[:, :, None] * ohf_i0[:, None, :])
    solo0_ref[...] = 1.0 - jnp.maximum(ohf_i0, ohf_j0)


def _make_sc_builder(B, S):
    info = plsc.get_sparse_core_info()
    NC, NS = info.num_cores, info.num_subcores
    NW = NC * NS
    assert B % NW == 0
    SPW = B // NW
    NCH = S // 16
    mesh = plsc.VectorSubcoreMesh(core_axis_name="c", subcore_axis_name="s")

    @functools.partial(
        pl.kernel,
        out_type=jax.ShapeDtypeStruct((B, S, S), jnp.float32),
        mesh=mesh,
        scratch_types=[
            pltpu.VMEM((S, S), jnp.float32),
            pltpu.VMEM((S, S), jnp.float32),
            pltpu.VMEM((S,), jnp.float32),
        ],
        compiler_params=pltpu.CompilerParams(needs_layout_passes=False),
    )
    def builder(dis_hbm, at0_hbm, solo0_hbm, at_hbm, dis_v, at_v, solo_v):
        wid = lax.axis_index("s") * NC + lax.axis_index("c")
        iota = lax.iota(jnp.int32, 16)
        cols = [iota + 16 * c for c in range(NCH)]
        perms = [iota ^ k for k in (8, 4, 2, 1)]
        INF = jnp.float32(jnp.inf)
        BIGI = jnp.int32(10 ** 9)

        def lanes_min(v):
            for p in perms:
                v = jnp.minimum(v, v.at[p].get(mode="promise_in_bounds"))
            return v

        def lanes_sum(v):
            for p in perms:
                v = v + v.at[p].get(mode="promise_in_bounds")
            return v

        lane_splats = [jnp.full((16,), l, dtype=jnp.int32) for l in range(16)]

        def per_sample(s, carry):
            b = wid * SPW + s
            pltpu.sync_copy(dis_hbm.at[b], dis_v)
            pltpu.sync_copy(at0_hbm.at[b], at_v)
            pltpu.sync_copy(solo0_hbm.at[b], solo_v)
            init = (tuple(solo_v[pl.ds(16 * c, 16)] for c in range(NCH)) +
                    (jnp.full((16,), 2.0),))

            def round_body(t, carry2):
                solo_ch = list(carry2[:NCH])
                ncv = carry2[NCH]
                solo_b = [v == 1.0 for v in solo_ch]
                nsv = jnp.maximum(jnp.float32(S) - ncv, 1.0)

                acc = [jnp.zeros((16,), jnp.float32)] * (2 * NCH)
                for cj_blk in range(NCH):
                    soloc = solo_ch[cj_blk]
                    for l in range(16):
                        j = 16 * cj_blk + l
                        sj = soloc.at[lane_splats[l]].get(
                            mode="promise_in_bounds")
                        cj = 1.0 - sj
                        for c in range(NCH):
                            row = dis_v[j, pl.ds(16 * c, 16)]
                            acc[c] = acc[c] + row * cj
                            acc[NCH + c] = acc[NCH + c] + row * sj
                ms2c = [jnp.where(solo_b[c], acc[c] / ncv, INF)
                        for c in range(NCH)]
                mc2s = [jnp.where(solo_b[c], INF, acc[NCH + c] / nsv)
                        for c in range(NCH)]

                def two_smallest(vch):
                    v1 = lanes_min(jnp.minimum(jnp.minimum(vch[0], vch[1]),
                                               jnp.minimum(vch[2], vch[3])))
                    cand = [jnp.where(vch[c] == v1, cols[c], BIGI)
                            for c in range(NCH)]
                    am = lanes_min(jnp.minimum(jnp.minimum(cand[0], cand[1]),
                                               jnp.minimum(cand[2], cand[3])))
                    masked = [jnp.where(cols[c] == am, INF, vch[c])
                              for c in range(NCH)]
                    v2 = lanes_min(jnp.minimum(
                        jnp.minimum(masked[0], masked[1]),
                        jnp.minimum(masked[2], masked[3])))
                    return v1, am, v2

                v1, ssp, v2 = two_smallest(ms2c)
                _, _, w2 = two_smallest(mc2s)
                tv = jnp.full((16,), t, dtype=jnp.int32)
                t1v = jnp.where(tv < S - 3, v2, v1)
                t2v = w2
                g1v = t1v < INF
                g2v = t2v < INF
                ones = jnp.ones((16,), jnp.float32)
                anyf = jnp.float32(0.0)
                for c in range(NCH):
                    d_c = plsc.load_gather(dis_v, [ssp, cols[c]])
                    e_row = ((d_c - t1v) < 0.0) & g1v & ~solo_b[c]
                    e_col = ((d_c - t2v) < 0.0) & g2v & ~solo_b[c]
                    plsc.store_scatter(at_v, [ssp, cols[c]], ones,
                                       mask=e_row)
                    plsc.store_scatter(at_v, [cols[c], ssp], ones,
                                       mask=e_col)
                    anyf = anyf + jnp.where(e_row | e_col, 1.0, 0.0)
                addf = jnp.where(lanes_sum(anyf) > 0.0, 1.0, 0.0)
                new_solo = tuple(
                    jnp.where(cols[c] == ssp, solo_ch[c] * (1.0 - addf),
                              solo_ch[c])
                    for c in range(NCH))
                return new_solo + (ncv + addf,)

            lax.fori_loop(0, S - 2, round_body, init)
            pltpu.sync_copy(at_v, at_hbm.at[b])
            return carry

        lax.fori_loop(0, SPW, per_sample, 0)

    return builder


def _out_kernel(at_ref, x_ref, W1_ref, W2_ref, out_ref):
    BB = at_ref.shape[0]
    W1 = W1_ref[...]
    W2 = W2_ref[...]
    for s in range(BB):
        acc = lax.dot_general(at_ref[s], x_ref[s], (((1,), (0,)), ((), ())),
                              preferred_element_type=jnp.float32)
        h = lax.dot_general(acc, W1, (((1,), (0,)), ((), ())),
                            preferred_element_type=jnp.float32)
        out_ref[s] = jnp.maximum(h + W2, 0.0)


def _halves(x, W1, W2, n_chunks):
    B, S, C = x.shape
    BB = 16
    BH = B // n_chunks
    builder = _make_sc_builder(BH, S)
    outs = []
    for h in range(n_chunks):
        xh = lax.slice_in_dim(x, h * BH, (h + 1) * BH, axis=0)
        dis, at0, solo0 = pl.pallas_call(
            _dist_kernel,
            grid=(BH // BB,),
            in_specs=[pl.BlockSpec((BB, S, C), lambda i: (i, 0, 0))],
            out_specs=[
                pl.BlockSpec((BB, S, S), lambda i: (i, 0, 0)),
                pl.BlockSpec((BB, S, S), lambda i: (i, 0, 0)),
                pl.BlockSpec((BB, S), lambda i: (i, 0)),
            ],
            out_shape=[
                jax.ShapeDtypeStruct((BH, S, S), jnp.float32),
                jax.ShapeDtypeStruct((BH, S, S), jnp.float32),
                jax.ShapeDtypeStruct((BH, S), jnp.float32),
            ],
        )(xh)
        at = builder(dis, at0, solo0)
        outs.append(pl.pallas_call(
            _out_kernel,
            grid=(BH // BB,),
            in_specs=[
                pl.BlockSpec((BB, S, S), lambda i: (i, 0, 0)),
                pl.BlockSpec((BB, S, C), lambda i: (i, 0, 0)),
                pl.BlockSpec((C, C), lambda i: (0, 0)),
                pl.BlockSpec((S, C), lambda i: (0, 0)),
            ],
            out_specs=pl.BlockSpec((BB, S, C), lambda i: (i, 0, 0)),
            out_shape=jax.ShapeDtypeStruct((BH, S, C), x.dtype),
        )(at, xh, W1, W2))
    return jnp.concatenate(outs, axis=0)


@jax.jit
def kernel(x, W1, W2):
    return _halves(x, W1, W2, 1)


# speedup vs baseline: 1.1368x; 1.0182x over previous
import functools

import jax
import jax.numpy as jnp
from jax import lax
from jax.experimental import pallas as pl
from jax.experimental.pallas import tpu as pltpu
from jax.experimental.pallas import tpu_sc as plsc


def _dist_kernel(x_ref, dis_ref, at0_ref, solo0_ref):
    x = x_ref[...]
    BB, S, C = x.shape
    INF = jnp.float32(jnp.inf)
    BIG = jnp.int32(10 ** 9)

    sq = jnp.sum(x * x, axis=-1)
    G = jnp.stack([
        lax.dot_general(x[s], x[s], (((1,), (1,)), ((), ())),
                        preferred_element_type=jnp.float32)
        for s in range(BB)
    ], axis=0)
    D2 = (sq[:, :, None] + sq[:, None, :]) - 2.0 * G
    Dis = jnp.sqrt(jnp.maximum(D2, 0.0))

    ii = lax.broadcasted_iota(jnp.int32, (BB, S), 1)
    eye = (lax.broadcasted_iota(jnp.int32, (S, S), 0) ==
           lax.broadcasted_iota(jnp.int32, (S, S), 1))

    m = jnp.where(eye[None], INF, Dis)
    rowmin = jnp.min(m, axis=2)
    gmin = jnp.min(rowmin, axis=1, keepdims=True)
    i0 = jnp.min(jnp.where(rowmin == gmin, ii, BIG), axis=1, keepdims=True)
    ohf_i0 = jnp.where(ii == i0, 1.0, 0.0)
    row_i0 = jnp.sum(jnp.where(ohf_i0[:, None, :] == 1.0, m, 0.0), axis=2)
    j0 = jnp.min(jnp.where(row_i0 == gmin, ii, BIG), axis=1, keepdims=True)
    ohf_j0 = jnp.where(ii == j0, 1.0, 0.0)

    dis_ref[...] = Dis
    at0_ref[...] = (ohf_i0[:, :, None] * ohf_j0[:, None, :] +
                    ohf_j0[:, :, None] * ohf_i0[:, None, :])
    solo0_ref[...] = 1.0 - jnp.maximum(ohf_i0, ohf_j0)


def _make_sc_builder(B, S):
    info = plsc.get_sparse_core_info()
    NC, NS = info.num_cores, info.num_subcores
    NW = NC * NS
    assert B % NW == 0
    SPW = B // NW
    NCH = S // 16
    mesh = plsc.VectorSubcoreMesh(core_axis_name="c", subcore_axis_name="s")

    @functools.partial(
        pl.kernel,
        out_type=jax.ShapeDtypeStruct((B, S, S), jnp.float32),
        mesh=mesh,
        scratch_types=[
            pltpu.VMEM((S, S), jnp.float32),
            pltpu.VMEM((S, S), jnp.float32),
            pltpu.VMEM((S,), jnp.float32),
        ],
        compiler_params=pltpu.CompilerParams(needs_layout_passes=False),
    )
    def builder(dis_hbm, at0_hbm, solo0_hbm, at_hbm, dis_v, at_v, solo_v):
        wid = lax.axis_index("s") * NC + lax.axis_index("c")
        iota = lax.iota(jnp.int32, 16)
        cols = [iota + 16 * c for c in range(NCH)]
        perms = [iota ^ k for k in (8, 4, 2, 1)]
        INF = jnp.float32(jnp.inf)
        BIGI = jnp.int32(10 ** 9)

        def lanes_min(v):
            for p in perms:
                v = jnp.minimum(v, v.at[p].get(mode="promise_in_bounds"))
            return v

        def lanes_sum(v):
            for p in perms:
                v = v + v.at[p].get(mode="promise_in_bounds")
            return v

        lane_splats = [jnp.full((16,), l, dtype=jnp.int32) for l in range(16)]

        def per_sample(s, carry):
            b = wid * SPW + s
            pltpu.sync_copy(dis_hbm.at[b], dis_v)
            pltpu.sync_copy(at0_hbm.at[b], at_v)
            pltpu.sync_copy(solo0_hbm.at[b], solo_v)
            init = (tuple(solo_v[pl.ds(16 * c, 16)] for c in range(NCH)) +
                    (jnp.full((16,), 2.0),))

            def round_body(t, carry2):
                solo_ch = list(carry2[:NCH])
                ncv = carry2[NCH]
                solo_b = [v == 1.0 for v in solo_ch]
                nsv = jnp.maximum(jnp.float32(S) - ncv, 1.0)

                acc = [jnp.zeros((16,), jnp.float32)] * (2 * NCH)
                for cj_blk in range(NCH):
                    soloc = solo_ch[cj_blk]
                    for l in range(16):
                        j = 16 * cj_blk + l
                        sj = soloc.at[lane_splats[l]].get(
                            mode="promise_in_bounds")
                        cj = 1.0 - sj
                        for c in range(NCH):
                            row = dis_v[j, pl.ds(16 * c, 16)]
                            acc[c] = acc[c] + row * cj
                            acc[NCH + c] = acc[NCH + c] + row * sj
                ms2c = [jnp.where(solo_b[c], acc[c] / ncv, INF)
                        for c in range(NCH)]
                mc2s = [jnp.where(solo_b[c], INF, acc[NCH + c] / nsv)
                        for c in range(NCH)]

                def two_smallest(vch):
                    v1 = lanes_min(jnp.minimum(jnp.minimum(vch[0], vch[1]),
                                               jnp.minimum(vch[2], vch[3])))
                    cand = [jnp.where(vch[c] == v1, cols[c], BIGI)
                            for c in range(NCH)]
                    am = lanes_min(jnp.minimum(jnp.minimum(cand[0], cand[1]),
                                               jnp.minimum(cand[2], cand[3])))
                    masked = [jnp.where(cols[c] == am, INF, vch[c])
                              for c in range(NCH)]
                    v2 = lanes_min(jnp.minimum(
                        jnp.minimum(masked[0], masked[1]),
                        jnp.minimum(masked[2], masked[3])))
                    return v1, am, v2

                v1, ssp, v2 = two_smallest(ms2c)
                _, _, w2 = two_smallest(mc2s)
                tv = jnp.full((16,), t, dtype=jnp.int32)
                t1v = jnp.where(tv < S - 3, v2, v1)
                t2v = w2
                g1v = t1v < INF
                g2v = t2v < INF
                ones = jnp.ones((16,), jnp.float32)
                anyf = jnp.float32(0.0)
                for c in range(NCH):
                    d_c = plsc.load_gather(dis_v, [ssp, cols[c]])
                    e_row = ((d_c - t1v) < 0.0) & g1v & ~solo_b[c]
                    e_col = ((d_c - t2v) < 0.0) & g2v & ~solo_b[c]
                    plsc.store_scatter(at_v, [ssp, cols[c]], ones,
                                       mask=e_row)
                    plsc.store_scatter(at_v, [cols[c], ssp], ones,
                                       mask=e_col)
                    anyf = anyf + jnp.where(e_row | e_col, 1.0, 0.0)
                addf = jnp.where(lanes_sum(anyf) > 0.0, 1.0, 0.0)
                new_solo = tuple(
                    jnp.where(cols[c] == ssp, solo_ch[c] * (1.0 - addf),
                              solo_ch[c])
                    for c in range(NCH))
                return new_solo + (ncv + addf,)

            lax.fori_loop(0, S - 2, round_body, init)
            pltpu.sync_copy(at_v, at_hbm.at[b])
            return carry

        lax.fori_loop(0, SPW, per_sample, 0)

    return builder


def _out_kernel(at_ref, x_ref, W1_ref, W2_ref, out_ref):
    BB = at_ref.shape[0]
    W1 = W1_ref[...]
    W2 = W2_ref[...]
    for s in range(BB):
        acc = lax.dot_general(at_ref[s], x_ref[s], (((1,), (0,)), ((), ())),
                              preferred_element_type=jnp.float32)
        h = lax.dot_general(acc, W1, (((1,), (0,)), ((), ())),
                            preferred_element_type=jnp.float32)
        out_ref[s] = jnp.maximum(h + W2, 0.0)


def _halves(x, W1, W2, n_chunks):
    B, S, C = x.shape
    BB = 32
    BH = B // n_chunks
    builder = _make_sc_builder(BH, S)
    outs = []
    for h in range(n_chunks):
        xh = lax.slice_in_dim(x, h * BH, (h + 1) * BH, axis=0)
        dis, at0, solo0 = pl.pallas_call(
            _dist_kernel,
            grid=(BH // BB,),
            in_specs=[pl.BlockSpec((BB, S, C), lambda i: (i, 0, 0))],
            out_specs=[
                pl.BlockSpec((BB, S, S), lambda i: (i, 0, 0)),
                pl.BlockSpec((BB, S, S), lambda i: (i, 0, 0)),
                pl.BlockSpec((BB, S), lambda i: (i, 0)),
            ],
            out_shape=[
                jax.ShapeDtypeStruct((BH, S, S), jnp.float32),
                jax.ShapeDtypeStruct((BH, S, S), jnp.float32),
                jax.ShapeDtypeStruct((BH, S), jnp.float32),
            ],
        )(xh)
        at = builder(dis, at0, solo0)
        outs.append(pl.pallas_call(
            _out_kernel,
            grid=(BH // BB,),
            in_specs=[
                pl.BlockSpec((BB, S, S), lambda i: (i, 0, 0)),
                pl.BlockSpec((BB, S, C), lambda i: (i, 0, 0)),
                pl.BlockSpec((C, C), lambda i: (0, 0)),
                pl.BlockSpec((S, C), lambda i: (0, 0)),
            ],
            out_specs=pl.BlockSpec((BB, S, C), lambda i: (i, 0, 0)),
            out_shape=jax.ShapeDtypeStruct((BH, S, C), x.dtype),
        )(at, xh, W1, W2))
    return jnp.concatenate(outs, axis=0)


@jax.jit
def kernel(x, W1, W2):
    return _halves(x, W1, W2, 1)
